# baseline (speedup 1.0000x reference)
import jax
import jax.numpy as jnp
from jax import lax
from jax.experimental import pallas as pl
from jax.experimental.pallas import tpu as pltpu
from jax.experimental.pallas import tpu_sc as plsc

B = 4096
L = 50
D = 64
W = D // 2
N = B * L
NC = 2
NS = 16
NW = NC * NS
N_W = N // NW
SUB = 128
CHUNK = 640
NSUB = CHUNK // SUB
NCHUNK = N_W // CHUNK
GROUPS = CHUNK // 16
IDX_ROWS = N_W // SUB


def _sc_body(uidx_hbm, midx_hbm, utab_hbm, mtab_hbm, out_hbm,
             uidx_v, midx_v, urows, mrows, out_v, sems):
    wid = lax.axis_index("s") * NC + lax.axis_index("c")

    pltpu.sync_copy(uidx_hbm.at[wid], uidx_v)
    pltpu.sync_copy(midx_hbm.at[wid], midx_v)

    iota = lax.iota(jnp.int32, 16)
    zero16 = jnp.zeros((16,), jnp.int32)
    himask = jnp.full((16,), -65536, jnp.int32)

    def issue(j, b):
        for k in range(NSUB):
            pltpu.async_copy(
                utab_hbm.at[uidx_v.at[j * NSUB + k]],
                urows.at[b].at[pl.ds(k * SUB, SUB)], sems.at[b, 0])
            pltpu.async_copy(
                mtab_hbm.at[midx_v.at[j * NSUB + k]],
                mrows.at[b].at[pl.ds(k * SUB, SUB)], sems.at[b, 1])

    def wait(j, b):
        for k in range(NSUB):
            pltpu.make_async_copy(
                utab_hbm.at[uidx_v.at[j * NSUB + k]],
                urows.at[b].at[pl.ds(k * SUB, SUB)], sems.at[b, 0]).wait()
            pltpu.make_async_copy(
                mtab_hbm.at[midx_v.at[j * NSUB + k]],
                mrows.at[b].at[pl.ds(k * SUB, SUB)], sems.at[b, 1]).wait()

    def compute(j, b):
        ub = urows.at[b]
        mb = mrows.at[b]

        def group_body(g, carry):
            base = (g * 16 + iota) * W
            accs = [jnp.zeros((16,), jnp.float32) for _ in range(4)]
            for w in range(W):
                idx = base + w
                uw = plsc.load_gather(ub, [zero16, idx])
                mw = plsc.load_gather(mb, [zero16, idx])
                ulo = plsc.bitcast(lax.shift_left(uw, 16), jnp.float32)
                mlo = plsc.bitcast(lax.shift_left(mw, 16), jnp.float32)
                uhi = plsc.bitcast(lax.bitwise_and(uw, himask), jnp.float32)
                mhi = plsc.bitcast(lax.bitwise_and(mw, himask), jnp.float32)
                accs[2 * (w % 2)] = accs[2 * (w % 2)] + ulo * mlo
                accs[2 * (w % 2) + 1] = accs[2 * (w % 2) + 1] + uhi * mhi
            acc = (accs[0] + accs[1]) + (accs[2] + accs[3])
            out_v[pl.ds(j * CHUNK + g * 16, 16)] = acc
            return carry

        lax.fori_loop(0, GROUPS, group_body, 0)

    issue(0, 0)
    issue(1, 1)

    def outer_body(i, carry):
        for b in range(2):
            j = 2 * i + b
            wait(j, b)
            compute(j, b)

            @pl.when(j + 2 < NCHUNK)
            def _():
                issue(j + 2, b)
        return carry

    lax.fori_loop(0, NCHUNK // 2, outer_body, 0, unroll=False)

    pltpu.sync_copy(out_v, out_hbm.at[pl.ds(wid * N_W, N_W)])


def _pack_bf16(table):
    t16 = table.astype(jnp.bfloat16).reshape(table.shape[0], W, 2)
    return lax.bitcast_convert_type(t16, jnp.int32)


def kernel(user_id, movie_title, user_table, movie_table):
    uidx = user_id.reshape(NW, IDX_ROWS, SUB)
    midx = movie_title.reshape(NW, IDX_ROWS, SUB)
    utab = _pack_bf16(user_table)
    mtab = _pack_bf16(movie_table)
    mesh = plsc.VectorSubcoreMesh(core_axis_name="c", subcore_axis_name="s")
    out = pl.kernel(
        _sc_body,
        out_type=jax.ShapeDtypeStruct((N,), jnp.float32),
        mesh=mesh,
        scratch_types=[
            pltpu.VMEM((IDX_ROWS, SUB), jnp.int32),
            pltpu.VMEM((IDX_ROWS, SUB), jnp.int32),
            pltpu.VMEM((2, CHUNK, W), jnp.int32),
            pltpu.VMEM((2, CHUNK, W), jnp.int32),
            pltpu.VMEM((N_W,), jnp.float32),
            pltpu.SemaphoreType.DMA((2, 2)),
        ],
        compiler_params=pltpu.CompilerParams(
            needs_layout_passes=False, use_tc_tiling_on_sc=False,
            disable_bounds_checks=True),
    )(uidx, midx, utab, mtab)
    return out.reshape(B, L)


# speedup vs baseline: 1.3614x; 1.3614x over previous
import jax
import jax.numpy as jnp
from jax import lax
from jax.experimental import pallas as pl
from jax.experimental.pallas import tpu as pltpu
from jax.experimental.pallas import tpu_sc as plsc

B = 4096
L = 50
D = 64
VOCAB = 100000
W = D // 2
N = B * L
NC = 2
NS = 16
NW = NC * NS
N_W = N // NW
SUB = 128
CHUNK = 640
NSUB = CHUNK // SUB
NCHUNK = N_W // CHUNK
GROUPS = CHUNK // 16
IDX_ROWS = N_W // SUB


def _sc_body(uidx_hbm, midx_hbm, utab_hbm, mtab_hbm, out_hbm,
             uidx_v, midx_v, urows, mrows, out_v, sems):
    wid = lax.axis_index("s") * NC + lax.axis_index("c")


    pltpu.sync_copy(uidx_hbm.at[wid], uidx_v)
    pltpu.sync_copy(midx_hbm.at[wid], midx_v)

    iota = lax.iota(jnp.int32, 16)
    zero16 = jnp.zeros((16,), jnp.int32)
    himask = jnp.full((16,), -65536, jnp.int32)

    def issue(j, b):
        for k in range(NSUB):
            pltpu.async_copy(
                utab_hbm.at[uidx_v.at[j * NSUB + k]],
                urows.at[b].at[pl.ds(k * SUB, SUB)], sems.at[b, 0])
            pltpu.async_copy(
                mtab_hbm.at[midx_v.at[j * NSUB + k]],
                mrows.at[b].at[pl.ds(k * SUB, SUB)], sems.at[b, 1])

    def wait(j, b):
        for k in range(NSUB):
            pltpu.make_async_copy(
                utab_hbm.at[uidx_v.at[j * NSUB + k]],
                urows.at[b].at[pl.ds(k * SUB, SUB)], sems.at[b, 0]).wait()
            pltpu.make_async_copy(
                mtab_hbm.at[midx_v.at[j * NSUB + k]],
                mrows.at[b].at[pl.ds(k * SUB, SUB)], sems.at[b, 1]).wait()

    def compute(j, b):
        ub = urows.at[b]
        mb = mrows.at[b]

        def group_body(g, carry):
            base = (g * 16 + iota) * W
            accs = [jnp.zeros((16,), jnp.float32) for _ in range(4)]
            for w in range(W):
                idx = base + w
                uw = plsc.load_gather(ub, [zero16, idx])
                mw = plsc.load_gather(mb, [zero16, idx])
                ulo = plsc.bitcast(lax.shift_left(uw, 16), jnp.float32)
                mlo = plsc.bitcast(lax.shift_left(mw, 16), jnp.float32)
                uhi = plsc.bitcast(lax.bitwise_and(uw, himask), jnp.float32)
                mhi = plsc.bitcast(lax.bitwise_and(mw, himask), jnp.float32)
                accs[2 * (w % 2)] = accs[2 * (w % 2)] + ulo * mlo
                accs[2 * (w % 2) + 1] = accs[2 * (w % 2) + 1] + uhi * mhi
            acc = (accs[0] + accs[1]) + (accs[2] + accs[3])
            out_v[pl.ds(j * CHUNK + g * 16, 16)] = acc
            return carry

        lax.fori_loop(0, GROUPS, group_body, 0)

    issue(0, 0)
    issue(1, 1)

    def outer_body(i, carry):
        for b in range(2):
            j = 2 * i + b
            wait(j, b)
            compute(j, b)

            @pl.when(j + 2 < NCHUNK)
            def _():
                issue(j + 2, b)
        return carry

    lax.fori_loop(0, NCHUNK // 2, outer_body, 0, unroll=False)

    pltpu.sync_copy(out_v, out_hbm.at[pl.ds(wid * N_W, N_W)])


def _pack_bf16(table):
    u = lax.bitcast_convert_type(table, jnp.uint32)
    r = u + jnp.uint32(0x7FFF) + ((u >> jnp.uint32(16)) & jnp.uint32(1))
    lo = r[:, :W] >> jnp.uint32(16)
    hi = r[:, W:] & jnp.uint32(0xFFFF0000)
    return lax.bitcast_convert_type(hi | lo, jnp.int32)


def kernel(user_id, movie_title, user_table, movie_table):
    uidx = user_id.reshape(NW, IDX_ROWS, SUB)
    midx = movie_title.reshape(NW, IDX_ROWS, SUB)
    utab = _pack_bf16(user_table)
    mtab = _pack_bf16(movie_table)
    mesh = plsc.VectorSubcoreMesh(core_axis_name="c", subcore_axis_name="s")
    out = pl.kernel(
        _sc_body,
        out_type=jax.ShapeDtypeStruct((N,), jnp.float32),
        mesh=mesh,
        scratch_types=[
            pltpu.VMEM((IDX_ROWS, SUB), jnp.int32),
            pltpu.VMEM((IDX_ROWS, SUB), jnp.int32),
            pltpu.VMEM((2, CHUNK, W), jnp.int32),
            pltpu.VMEM((2, CHUNK, W), jnp.int32),
            pltpu.VMEM((N_W,), jnp.float32),
            pltpu.SemaphoreType.DMA((2, 2)),
        ],
        compiler_params=pltpu.CompilerParams(
            needs_layout_passes=False, use_tc_tiling_on_sc=False,
            disable_bounds_checks=True),
    )(uidx, midx, utab, mtab)
    return out.reshape(B, L)


# speedup vs baseline: 1.3622x; 1.0005x over previous
import jax
import jax.numpy as jnp
from jax import lax
from jax.experimental import pallas as pl
from jax.experimental.pallas import tpu as pltpu
from jax.experimental.pallas import tpu_sc as plsc

B = 4096
L = 50
D = 64
VOCAB = 100000
W = D // 2
N = B * L
NC = 2
NS = 16
NW = NC * NS
N_W = N // NW
SUB = 128
CHUNK = 640
NSUB = CHUNK // SUB
NCHUNK = N_W // CHUNK
GROUPS = CHUNK // 16
IDX_ROWS = N_W // SUB


def _sc_body(uidx_hbm, midx_hbm, utab_hbm, mtab_hbm, out_hbm,
             uidx_v, midx_v, urows, mrows, out_v, sems):
    wid = lax.axis_index("s") * NC + lax.axis_index("c")

    utab32 = utab_hbm
    mtab32 = mtab_hbm

    pltpu.sync_copy(uidx_hbm.at[wid], uidx_v)
    pltpu.sync_copy(midx_hbm.at[wid], midx_v)

    iota = lax.iota(jnp.int32, 16)
    zero16 = jnp.zeros((16,), jnp.int32)
    himask = jnp.full((16,), -65536, jnp.int32)

    def _dst(rows, b, k):
        return rows.at[b].at[pl.ds(k * SUB, SUB)]


    def issue(j, b):
        for k in range(NSUB):
            pltpu.async_copy(
                utab32.at[uidx_v.at[j * NSUB + k]], _dst(urows, b, k),
                sems.at[b, 0])
            pltpu.async_copy(
                mtab32.at[midx_v.at[j * NSUB + k]], _dst(mrows, b, k),
                sems.at[b, 1])

    def wait(j, b):
        for k in range(NSUB):
            pltpu.make_async_copy(
                utab32.at[uidx_v.at[j * NSUB + k]], _dst(urows, b, k),
                sems.at[b, 0]).wait()
            pltpu.make_async_copy(
                mtab32.at[midx_v.at[j * NSUB + k]], _dst(mrows, b, k),
                sems.at[b, 1]).wait()

    def compute(j, b):
        ub = urows.at[b]
        mb = mrows.at[b]

        def group_body(g, carry):
            base = (g * 16 + iota) * W
            accs = [jnp.zeros((16,), jnp.float32) for _ in range(4)]
            for w in range(W):
                idx = base + w
                uw = plsc.load_gather(ub, [zero16, idx])
                mw = plsc.load_gather(mb, [zero16, idx])
                ulo = plsc.bitcast(lax.shift_left(uw, 16), jnp.float32)
                mlo = plsc.bitcast(lax.shift_left(mw, 16), jnp.float32)
                uhi = plsc.bitcast(lax.bitwise_and(uw, himask), jnp.float32)
                mhi = plsc.bitcast(lax.bitwise_and(mw, himask), jnp.float32)
                accs[2 * (w % 2)] = accs[2 * (w % 2)] + ulo * mlo
                accs[2 * (w % 2) + 1] = accs[2 * (w % 2) + 1] + uhi * mhi
            acc = (accs[0] + accs[1]) + (accs[2] + accs[3])
            out_v[pl.ds(j * CHUNK + g * 16, 16)] = acc
            return carry

        lax.fori_loop(0, GROUPS, group_body, 0)

    issue(0, 0)
    issue(1, 1)

    def outer_body(i, carry):
        for b in range(2):
            j = 2 * i + b
            wait(j, b)
            compute(j, b)

            @pl.when(j + 2 < NCHUNK)
            def _():
                issue(j + 2, b)
        return carry

    lax.fori_loop(0, NCHUNK // 2, outer_body, 0, unroll=False)

    pltpu.sync_copy(out_v, out_hbm.at[pl.ds(wid * N_W, N_W)])


def _pack_bf16(table):
    u = lax.bitcast_convert_type(table, jnp.uint32)
    r = u + jnp.uint32(0x7FFF) + ((u >> jnp.uint32(16)) & jnp.uint32(1))
    lo = r[:, :W] >> jnp.uint32(16)
    hi = r[:, W:] & jnp.uint32(0xFFFF0000)
    return lax.bitcast_convert_type(hi | lo, jnp.int32)


def kernel(user_id, movie_title, user_table, movie_table):
    uidx = user_id.reshape(NW, IDX_ROWS, SUB)
    midx = movie_title.reshape(NW, IDX_ROWS, SUB)
    utab = _pack_bf16(user_table)
    mtab = _pack_bf16(movie_table)
    mesh = plsc.VectorSubcoreMesh(core_axis_name="c", subcore_axis_name="s")
    out = pl.kernel(
        _sc_body,
        out_type=jax.ShapeDtypeStruct((N,), jnp.float32),
        mesh=mesh,
        scratch_types=[
            pltpu.VMEM((IDX_ROWS, SUB), jnp.int32),
            pltpu.VMEM((IDX_ROWS, SUB), jnp.int32),
            pltpu.VMEM((2, CHUNK, W), jnp.int32),
            pltpu.VMEM((2, CHUNK, W), jnp.int32),
            pltpu.VMEM((N_W,), jnp.float32),
            pltpu.SemaphoreType.DMA((2, 2)),
        ],
        compiler_params=pltpu.CompilerParams(
            needs_layout_passes=False, use_tc_tiling_on_sc=False,
            disable_bounds_checks=True),
    )(uidx, midx, utab, mtab)
    return out.reshape(B, L)
